# baseline (speedup 1.0000x reference)
import functools

import jax
import jax.numpy as jnp
from jax import lax
from jax.experimental import pallas as pl
from jax.experimental.pallas import tpu as pltpu
from jax.experimental.pallas import tpu_sc as plsc

NC = 2
NS = 16
NW = NC * NS
B = 256
ZB = 128
LANES = 16
FH = 64


def _sc_mesh():
  return plsc.VectorSubcoreMesh(core_axis_name="c", subcore_axis_name="s")


_SC_PARAMS = pltpu.CompilerParams(use_tc_tiling_on_sc=False)


def _deg_hist(cols3, npad, chunks):
  rows_per_sub = npad // NS

  @functools.partial(
      pl.kernel,
      out_type=jax.ShapeDtypeStruct((NC, npad, LANES), jnp.float32),
      mesh=_sc_mesh(),
      scratch_types=[
          pltpu.VMEM((chunks, B), jnp.int32),
          pltpu.VMEM((B, LANES), jnp.float32),
          pltpu.VMEM_SHARED((npad, LANES), jnp.float32),
      ],
      compiler_params=_SC_PARAMS,
  )
  def k(col_hbm, out_hbm, col_v, ones_v, acc):
    c = lax.axis_index("c")
    s = lax.axis_index("s")
    wid = s * NC + c
    pltpu.sync_copy(col_hbm.at[wid], col_v)

    @pl.loop(0, B)
    def _(i):
      ones_v[i, pl.ds(0, LANES)] = jnp.zeros((LANES,), jnp.float32)

    @pl.loop(0, rows_per_sub, step=ZB)
    def _(r):
      pltpu.sync_copy(ones_v.at[pl.ds(0, ZB)],
                      acc.at[pl.ds(s * rows_per_sub + r, ZB)])

    @pl.loop(0, B)
    def _(i):
      ones_v[i, pl.ds(0, LANES)] = jnp.ones((LANES,), jnp.float32)

    plsc.subcore_barrier()

    @pl.loop(0, chunks)
    def _(j):
      pltpu.sync_copy(ones_v, acc.at[col_v.at[j]], add=True)

    plsc.subcore_barrier()
    sl = pl.ds(s * rows_per_sub, rows_per_sub)
    pltpu.sync_copy(acc.at[sl], out_hbm.at[c].at[sl])

  return k(cols3)


def _edge_agg(gh, rows3, cols3, npad, passes, chunks):
  rows_per_sub = npad // NS

  @functools.partial(
      pl.kernel,
      out_type=jax.ShapeDtypeStruct((NC, passes, npad, FH), jnp.float32),
      mesh=_sc_mesh(),
      scratch_types=[
          pltpu.VMEM((chunks, B), jnp.int32),
          pltpu.VMEM((chunks, B), jnp.int32),
          pltpu.VMEM((B, FH), jnp.float32),
          pltpu.VMEM((ZB, FH), jnp.float32),
          pltpu.VMEM_SHARED((npad, FH), jnp.float32),
          pltpu.VMEM_SHARED((npad, FH), jnp.float32),
          pltpu.SemaphoreType.DMA,
      ],
      compiler_params=_SC_PARAMS,
  )
  def k(g_hbm, row_hbm, col_hbm, out_hbm, row_v, col_v, buf, zbuf, table,
        acc, sem):
    c = lax.axis_index("c")
    s = lax.axis_index("s")
    wid = s * NC + c
    sl = pl.ds(s * rows_per_sub, rows_per_sub)

    pltpu.sync_copy(row_hbm.at[wid], row_v)
    pltpu.sync_copy(col_hbm.at[wid], col_v)

    @pl.loop(0, ZB)
    def _(i):
      for j in range(FH // LANES):
        zbuf[i, pl.ds(j * LANES, LANES)] = jnp.zeros((LANES,), jnp.float32)

    for p in range(passes):
      pltpu.sync_copy(g_hbm.at[p].at[sl], table.at[sl])

      @pl.loop(0, rows_per_sub, step=ZB)
      def _(r):
        pltpu.sync_copy(zbuf, acc.at[pl.ds(s * rows_per_sub + r, ZB)])

      plsc.subcore_barrier()

      @pl.loop(0, chunks)
      def _(j):
        pltpu.async_copy(table.at[row_v.at[j]], buf, sem).wait()
        pltpu.sync_copy(buf, acc.at[col_v.at[j]], add=True)

      plsc.subcore_barrier()
      pltpu.sync_copy(acc.at[sl], out_hbm.at[c].at[p].at[sl])
      plsc.subcore_barrier()

  return k(gh, rows3, cols3)


def _tc_matmul(xp, w):
  m, kdim = xp.shape
  f = w.shape[1]
  bm = 1024

  def body(x_ref, w_ref, o_ref):
    o_ref[...] = jnp.dot(x_ref[...], w_ref[...],
                         preferred_element_type=jnp.float32)

  return pl.pallas_call(
      body,
      grid=(m // bm,),
      in_specs=[pl.BlockSpec((bm, kdim), lambda i: (i, 0)),
                pl.BlockSpec((kdim, f), lambda i: (0, 0))],
      out_specs=pl.BlockSpec((bm, f), lambda i: (i, 0)),
      out_shape=jax.ShapeDtypeStruct((m, f), jnp.float32),
  )(xp, w)


def _scale(h1, da, db, n_real):
  m, h = h1.shape
  bm = 1024
  halves = h // FH

  def body(h_ref, da_ref, db_ref, dinv_ref, g_ref):
    i = pl.program_id(0)
    rowid = lax.broadcasted_iota(jnp.int32, (bm, 1), 0) + i * bm
    deg = (da_ref[:, 0:1] + db_ref[:, 0:1]
           + jnp.where(rowid < n_real, 1.0, 0.0))
    dinv = jnp.where(deg > 0, lax.rsqrt(jnp.maximum(deg, 1e-12)), 0.0)
    dinv_ref[...] = jnp.broadcast_to(dinv, (bm, LANES))
    g = dinv * h_ref[...]
    for p in range(halves):
      g_ref[p, :, :] = g[:, p * FH:(p + 1) * FH]

  return pl.pallas_call(
      body,
      grid=(m // bm,),
      in_specs=[pl.BlockSpec((bm, h), lambda i: (i, 0)),
                pl.BlockSpec((bm, LANES), lambda i: (i, 0)),
                pl.BlockSpec((bm, LANES), lambda i: (i, 0))],
      out_specs=[pl.BlockSpec((bm, LANES), lambda i: (i, 0)),
                 pl.BlockSpec((halves, bm, FH), lambda i: (0, i, 0))],
      out_shape=[jax.ShapeDtypeStruct((m, LANES), jnp.float32),
                 jax.ShapeDtypeStruct((halves, m, FH), jnp.float32)],
  )(h1, da, db)


def _layer2_in(s1, g1h, dinv, b1r, w2):
  nc, halves, m, fh = s1.shape
  h = halves * fh
  c = w2.shape[1]
  bm = 1024

  def body(s_ref, g_ref, dv, b, w, g2_ref):
    d = dv[:, 0:1]
    zs = []
    for p in range(halves):
      t = s_ref[0, p, :, :] + s_ref[1, p, :, :] + g_ref[p, :, :]
      zs.append(jnp.maximum(d * t + b[0:1, p * fh:(p + 1) * fh], 0.0))
    z = jnp.concatenate(zs, axis=1)
    g2_ref[0, :, :] = d * jnp.dot(z, w[...],
                                  preferred_element_type=jnp.float32)

  return pl.pallas_call(
      body,
      grid=(m // bm,),
      in_specs=[pl.BlockSpec((nc, halves, bm, fh), lambda i: (0, 0, i, 0)),
                pl.BlockSpec((halves, bm, fh), lambda i: (0, i, 0)),
                pl.BlockSpec((bm, LANES), lambda i: (i, 0)),
                pl.BlockSpec((1, h), lambda i: (0, 0)),
                pl.BlockSpec((h, c), lambda i: (0, 0))],
      out_specs=pl.BlockSpec((1, bm, c), lambda i: (0, i, 0)),
      out_shape=jax.ShapeDtypeStruct((1, m, c), jnp.float32),
  )(s1, g1h, dinv, b1r, w2)


def _finish(s2, g2h, dinv, b2r):
  nc, _, m, c = s2.shape
  bm = 1024

  def body(s_ref, g_ref, dv, b, o_ref):
    d = dv[:, 0:1]
    o_ref[...] = (d * (s_ref[0, 0, :, :] + s_ref[1, 0, :, :]
                       + g_ref[0, :, :]) + b[0:1, :])

  return pl.pallas_call(
      body,
      grid=(m // bm,),
      in_specs=[pl.BlockSpec((nc, 1, bm, c), lambda i: (0, 0, i, 0)),
                pl.BlockSpec((1, bm, c), lambda i: (0, i, 0)),
                pl.BlockSpec((bm, LANES), lambda i: (i, 0)),
                pl.BlockSpec((1, c), lambda i: (0, 0))],
      out_specs=pl.BlockSpec((bm, c), lambda i: (i, 0)),
      out_shape=jax.ShapeDtypeStruct((m, c), jnp.float32),
  )(s2, g2h, dinv, b2r)


def _ceil_to(v, mult):
  return (v + mult - 1) // mult * mult


@jax.jit
def kernel(x, adjs, W1, b1, W2, b2):
  n, _ = x.shape
  h = W1.shape[1]
  c = W2.shape[1]
  e = adjs.shape[1]

  npad = _ceil_to(n + 1, NS * ZB)
  chunks = _ceil_to(e, NW * B) // (NW * B)
  epad = chunks * NW * B

  row = adjs[0].astype(jnp.int32)
  col = adjs[1].astype(jnp.int32)
  pad_idx = jnp.full((epad - e,), n, jnp.int32)
  rows3 = jnp.concatenate([row, pad_idx]).reshape(NW, chunks, B)
  cols3 = jnp.concatenate([col, pad_idx]).reshape(NW, chunks, B)
  xp = jnp.pad(x, ((0, npad - n), (0, 0)))

  degp = _deg_hist(cols3, npad, chunks)
  h1 = _tc_matmul(xp, W1)
  dinv, g1h = _scale(h1, degp[0], degp[1], n)
  s1 = _edge_agg(g1h, rows3, cols3, npad, h // FH, chunks)
  g2h = _layer2_in(s1, g1h, dinv, b1.reshape(1, h), W2)
  s2 = _edge_agg(g2h, rows3, cols3, npad, c // FH, chunks)
  outp = _finish(s2, g2h, dinv, b2.reshape(1, c))
  return outp[:n]


# speedup vs baseline: 1.0308x; 1.0308x over previous
import functools

import jax
import jax.numpy as jnp
from jax import lax
from jax.experimental import pallas as pl
from jax.experimental.pallas import tpu as pltpu
from jax.experimental.pallas import tpu_sc as plsc

NC = 2
NS = 16
NW = NC * NS
B = 128
ZB = 128
LANES = 16
FH = 64


def _sc_mesh():
  return plsc.VectorSubcoreMesh(core_axis_name="c", subcore_axis_name="s")


_SC_PARAMS = pltpu.CompilerParams(use_tc_tiling_on_sc=False)


def _deg_hist(rc, npad, chunks):
  rows_per_sub = npad // NS

  @functools.partial(
      pl.kernel,
      out_type=jax.ShapeDtypeStruct((NC, npad, LANES), jnp.float32),
      mesh=_sc_mesh(),
      scratch_types=[
          pltpu.VMEM((chunks, B), jnp.int32),
          pltpu.VMEM((B, LANES), jnp.float32),
          pltpu.VMEM_SHARED((npad, LANES), jnp.float32),
      ],
      compiler_params=_SC_PARAMS,
  )
  def k(rc_hbm, out_hbm, col_v, ones_v, acc):
    c = lax.axis_index("c")
    s = lax.axis_index("s")
    wid = s * NC + c
    pltpu.sync_copy(rc_hbm.at[1].at[wid], col_v)

    @pl.loop(0, B)
    def _(i):
      ones_v[i, pl.ds(0, LANES)] = jnp.zeros((LANES,), jnp.float32)

    @pl.loop(0, rows_per_sub, step=ZB)
    def _(r):
      pltpu.sync_copy(ones_v.at[pl.ds(0, ZB)],
                      acc.at[pl.ds(s * rows_per_sub + r, ZB)])

    @pl.loop(0, B)
    def _(i):
      ones_v[i, pl.ds(0, LANES)] = jnp.ones((LANES,), jnp.float32)

    plsc.subcore_barrier()

    @pl.loop(0, chunks)
    def _(j):
      pltpu.sync_copy(ones_v, acc.at[col_v.at[j]], add=True)

    plsc.subcore_barrier()
    sl = pl.ds(s * rows_per_sub, rows_per_sub)
    pltpu.sync_copy(acc.at[sl], out_hbm.at[c].at[sl])

  return k(rc)


def _edge_agg(gh, rc, npad, passes, chunks):
  rows_per_sub = npad // NS

  @functools.partial(
      pl.kernel,
      out_type=jax.ShapeDtypeStruct((NC, passes, npad, FH), jnp.float32),
      mesh=_sc_mesh(),
      scratch_types=[
          pltpu.VMEM((chunks, B), jnp.int32),
          pltpu.VMEM((chunks, B), jnp.int32),
          pltpu.VMEM((B, FH), jnp.float32),
          pltpu.VMEM((ZB, FH), jnp.float32),
          pltpu.VMEM_SHARED((npad, FH), jnp.float32),
          pltpu.VMEM_SHARED((npad, FH), jnp.float32),
          pltpu.SemaphoreType.DMA,
      ],
      compiler_params=_SC_PARAMS,
  )
  def k(g_hbm, rc_hbm, out_hbm, row_v, col_v, buf, zbuf, table,
        acc, sem):
    c = lax.axis_index("c")
    s = lax.axis_index("s")
    wid = s * NC + c
    sl = pl.ds(s * rows_per_sub, rows_per_sub)

    pltpu.sync_copy(rc_hbm.at[0].at[wid], row_v)
    pltpu.sync_copy(rc_hbm.at[1].at[wid], col_v)

    @pl.loop(0, ZB)
    def _(i):
      for j in range(FH // LANES):
        zbuf[i, pl.ds(j * LANES, LANES)] = jnp.zeros((LANES,), jnp.float32)

    for p in range(passes):
      pltpu.sync_copy(g_hbm.at[p].at[sl], table.at[sl])

      @pl.loop(0, rows_per_sub, step=ZB)
      def _(r):
        pltpu.sync_copy(zbuf, acc.at[pl.ds(s * rows_per_sub + r, ZB)])

      plsc.subcore_barrier()

      @pl.loop(0, chunks)
      def _(j):
        pltpu.async_copy(table.at[row_v.at[j]], buf, sem).wait()
        pltpu.sync_copy(buf, acc.at[col_v.at[j]], add=True)

      plsc.subcore_barrier()
      pltpu.sync_copy(acc.at[sl], out_hbm.at[c].at[p].at[sl])
      plsc.subcore_barrier()

  return k(gh, rc)


def _mm_scale(xp, w, degp, n_real):
  m, kdim = xp.shape
  h = w.shape[1]
  bm = 1024
  halves = h // FH

  def body(x_ref, w_ref, dg_ref, dinv_ref, g_ref):
    i = pl.program_id(0)
    rowid = lax.broadcasted_iota(jnp.int32, (bm, 1), 0) + i * bm
    deg = (dg_ref[0, :, 0:1] + dg_ref[1, :, 0:1]
           + jnp.where(rowid < n_real, 1.0, 0.0))
    dinv = jnp.where(deg > 0, lax.rsqrt(jnp.maximum(deg, 1e-12)), 0.0)
    dinv_ref[...] = jnp.broadcast_to(dinv, (bm, LANES))
    g = dinv * jnp.dot(x_ref[...], w_ref[...],
                       preferred_element_type=jnp.float32)
    for p in range(halves):
      g_ref[p, :, :] = g[:, p * FH:(p + 1) * FH]

  return pl.pallas_call(
      body,
      grid=(m // bm,),
      in_specs=[pl.BlockSpec((bm, kdim), lambda i: (i, 0)),
                pl.BlockSpec((kdim, h), lambda i: (0, 0)),
                pl.BlockSpec((NC, bm, LANES), lambda i: (0, i, 0))],
      out_specs=[pl.BlockSpec((bm, LANES), lambda i: (i, 0)),
                 pl.BlockSpec((halves, bm, FH), lambda i: (0, i, 0))],
      out_shape=[jax.ShapeDtypeStruct((m, LANES), jnp.float32),
                 jax.ShapeDtypeStruct((halves, m, FH), jnp.float32)],
  )(xp, w, degp)


def _layer2_in(s1, g1h, dinv, b1r, w2):
  nc, halves, m, fh = s1.shape
  h = halves * fh
  c = w2.shape[1]
  bm = 1024

  def body(s_ref, g_ref, dv, b, w, g2_ref):
    d = dv[:, 0:1]
    zs = []
    for p in range(halves):
      t = s_ref[0, p, :, :] + s_ref[1, p, :, :] + g_ref[p, :, :]
      zs.append(jnp.maximum(d * t + b[0:1, p * fh:(p + 1) * fh], 0.0))
    z = jnp.concatenate(zs, axis=1)
    g2_ref[0, :, :] = d * jnp.dot(z, w[...],
                                  preferred_element_type=jnp.float32)

  return pl.pallas_call(
      body,
      grid=(m // bm,),
      in_specs=[pl.BlockSpec((nc, halves, bm, fh), lambda i: (0, 0, i, 0)),
                pl.BlockSpec((halves, bm, fh), lambda i: (0, i, 0)),
                pl.BlockSpec((bm, LANES), lambda i: (i, 0)),
                pl.BlockSpec((1, h), lambda i: (0, 0)),
                pl.BlockSpec((h, c), lambda i: (0, 0))],
      out_specs=pl.BlockSpec((1, bm, c), lambda i: (0, i, 0)),
      out_shape=jax.ShapeDtypeStruct((1, m, c), jnp.float32),
  )(s1, g1h, dinv, b1r, w2)


def _finish(s2, g2h, dinv, b2r, n_out):
  nc, _, m, c = s2.shape
  bm = 400 if n_out % 400 == 0 else None

  def body(s_ref, g_ref, dv, b, o_ref):
    d = dv[:, 0:1]
    o_ref[...] = (d * (s_ref[0, 0, :, :] + s_ref[1, 0, :, :]
                       + g_ref[0, :, :]) + b[0:1, :])

  if bm is None:
    bm, rows = 1024, m
  else:
    rows = n_out
  out = pl.pallas_call(
      body,
      grid=(rows // bm,),
      in_specs=[pl.BlockSpec((nc, 1, bm, c), lambda i: (0, 0, i, 0)),
                pl.BlockSpec((1, bm, c), lambda i: (0, i, 0)),
                pl.BlockSpec((bm, LANES), lambda i: (i, 0)),
                pl.BlockSpec((1, c), lambda i: (0, 0))],
      out_specs=pl.BlockSpec((bm, c), lambda i: (i, 0)),
      out_shape=jax.ShapeDtypeStruct((rows, c), jnp.float32),
  )(s2, g2h, dinv, b2r)
  return out[:n_out]


def _ceil_to(v, mult):
  return (v + mult - 1) // mult * mult


@jax.jit
def kernel(x, adjs, W1, b1, W2, b2):
  n, _ = x.shape
  h = W1.shape[1]
  c = W2.shape[1]
  e = adjs.shape[1]

  npad = _ceil_to(n + 1, NS * ZB)
  chunks = _ceil_to(e, NW * B) // (NW * B)
  epad = chunks * NW * B

  pad_idx = jnp.full((2, epad - e), n, jnp.int32)
  rc = jnp.concatenate([adjs.astype(jnp.int32), pad_idx],
                       axis=1).reshape(2, NW, chunks, B)
  xp = jnp.pad(x, ((0, npad - n), (0, 0)))

  degp = _deg_hist(rc, npad, chunks)
  dinv, g1h = _mm_scale(xp, W1, degp, n)
  s1 = _edge_agg(g1h, rc, npad, h // FH, chunks)
  g2h = _layer2_in(s1, g1h, dinv, b1.reshape(1, h), W2)
  s2 = _edge_agg(g2h, rc, npad, c // FH, chunks)
  return _finish(s2, g2h, dinv, b2.reshape(1, c), n)


# speedup vs baseline: 1.0604x; 1.0287x over previous
import functools

import jax
import jax.numpy as jnp
from jax import lax
from jax.experimental import pallas as pl
from jax.experimental.pallas import tpu as pltpu
from jax.experimental.pallas import tpu_sc as plsc

NC = 2
NS = 16
NW = NC * NS
B = 128
ZB = 128
LANES = 16
FH = 64


def _sc_mesh():
  return plsc.VectorSubcoreMesh(core_axis_name="c", subcore_axis_name="s")


_SC_PARAMS = pltpu.CompilerParams(use_tc_tiling_on_sc=False)


def _deg_hist(rc, npad, chunks):
  rows_per_sub = npad // NS

  @functools.partial(
      pl.kernel,
      out_type=jax.ShapeDtypeStruct((NC, npad, LANES), jnp.float32),
      mesh=_sc_mesh(),
      scratch_types=[
          pltpu.VMEM((chunks, B), jnp.int32),
          pltpu.VMEM((B, LANES), jnp.float32),
          pltpu.VMEM_SHARED((npad, LANES), jnp.float32),
      ],
      compiler_params=_SC_PARAMS,
  )
  def k(rc_hbm, out_hbm, col_v, ones_v, acc):
    c = lax.axis_index("c")
    s = lax.axis_index("s")
    wid = s * NC + c
    pltpu.sync_copy(rc_hbm.at[1].at[wid], col_v)

    @pl.loop(0, B)
    def _(i):
      ones_v[i, pl.ds(0, LANES)] = jnp.zeros((LANES,), jnp.float32)

    @pl.loop(0, rows_per_sub, step=ZB)
    def _(r):
      pltpu.sync_copy(ones_v.at[pl.ds(0, ZB)],
                      acc.at[pl.ds(s * rows_per_sub + r, ZB)])

    @pl.loop(0, B)
    def _(i):
      ones_v[i, pl.ds(0, LANES)] = jnp.ones((LANES,), jnp.float32)

    plsc.subcore_barrier()

    @pl.loop(0, chunks)
    def _(j):
      pltpu.sync_copy(ones_v, acc.at[col_v.at[j]], add=True)

    plsc.subcore_barrier()
    sl = pl.ds(s * rows_per_sub, rows_per_sub)
    pltpu.sync_copy(acc.at[sl], out_hbm.at[c].at[sl])

  return k(rc)


def _edge_agg_split(gh, rc, npad, chunks):
  rows_per_sub = npad // NS

  @functools.partial(
      pl.kernel,
      out_type=jax.ShapeDtypeStruct((NC, npad, FH), jnp.float32),
      mesh=_sc_mesh(),
      scratch_types=[
          pltpu.VMEM((chunks, B), jnp.int32),
          pltpu.VMEM((chunks, B), jnp.int32),
          pltpu.VMEM((B, FH), jnp.float32),
          pltpu.VMEM((ZB, FH), jnp.float32),
          pltpu.VMEM_SHARED((npad, FH), jnp.float32),
          pltpu.VMEM_SHARED((npad, FH), jnp.float32),
          pltpu.SemaphoreType.DMA,
      ],
      compiler_params=_SC_PARAMS,
  )
  def k(g_hbm, rc_hbm, out_hbm, row_v, col_v, buf, zbuf, table, acc, sem):
    c = lax.axis_index("c")
    s = lax.axis_index("s")
    sl = pl.ds(s * rows_per_sub, rows_per_sub)

    @pl.loop(0, ZB)
    def _(i):
      for j in range(FH // LANES):
        zbuf[i, pl.ds(j * LANES, LANES)] = jnp.zeros((LANES,), jnp.float32)

    pltpu.sync_copy(g_hbm.at[c].at[sl], table.at[sl])

    @pl.loop(0, rows_per_sub, step=ZB)
    def _(r):
      pltpu.sync_copy(zbuf, acc.at[pl.ds(s * rows_per_sub + r, ZB)])

    plsc.subcore_barrier()

    for w in range(2):
      wid = 2 * s + w
      pltpu.sync_copy(rc_hbm.at[0].at[wid], row_v)
      pltpu.sync_copy(rc_hbm.at[1].at[wid], col_v)

      @pl.loop(0, chunks)
      def _(j):
        pltpu.async_copy(table.at[row_v.at[j]], buf, sem).wait()
        pltpu.sync_copy(buf, acc.at[col_v.at[j]], add=True)

    plsc.subcore_barrier()
    pltpu.sync_copy(acc.at[sl], out_hbm.at[c].at[sl])

  return k(gh, rc)


def _edge_agg(gh, rc, npad, passes, chunks):
  rows_per_sub = npad // NS

  @functools.partial(
      pl.kernel,
      out_type=jax.ShapeDtypeStruct((NC, passes, npad, FH), jnp.float32),
      mesh=_sc_mesh(),
      scratch_types=[
          pltpu.VMEM((chunks, B), jnp.int32),
          pltpu.VMEM((chunks, B), jnp.int32),
          pltpu.VMEM((B, FH), jnp.float32),
          pltpu.VMEM((ZB, FH), jnp.float32),
          pltpu.VMEM_SHARED((npad, FH), jnp.float32),
          pltpu.VMEM_SHARED((npad, FH), jnp.float32),
          pltpu.SemaphoreType.DMA,
      ],
      compiler_params=_SC_PARAMS,
  )
  def k(g_hbm, rc_hbm, out_hbm, row_v, col_v, buf, zbuf, table,
        acc, sem):
    c = lax.axis_index("c")
    s = lax.axis_index("s")
    wid = s * NC + c
    sl = pl.ds(s * rows_per_sub, rows_per_sub)

    pltpu.sync_copy(rc_hbm.at[0].at[wid], row_v)
    pltpu.sync_copy(rc_hbm.at[1].at[wid], col_v)

    @pl.loop(0, ZB)
    def _(i):
      for j in range(FH // LANES):
        zbuf[i, pl.ds(j * LANES, LANES)] = jnp.zeros((LANES,), jnp.float32)

    for p in range(passes):
      pltpu.sync_copy(g_hbm.at[p].at[sl], table.at[sl])

      @pl.loop(0, rows_per_sub, step=ZB)
      def _(r):
        pltpu.sync_copy(zbuf, acc.at[pl.ds(s * rows_per_sub + r, ZB)])

      plsc.subcore_barrier()

      @pl.loop(0, chunks)
      def _(j):
        pltpu.async_copy(table.at[row_v.at[j]], buf, sem).wait()
        pltpu.sync_copy(buf, acc.at[col_v.at[j]], add=True)

      plsc.subcore_barrier()
      pltpu.sync_copy(acc.at[sl], out_hbm.at[c].at[p].at[sl])
      plsc.subcore_barrier()

  return k(gh, rc)


def _mm_scale(xp, w, degp, n_real):
  m, kdim = xp.shape
  h = w.shape[1]
  bm = 1024
  halves = h // FH

  def body(x_ref, w_ref, dg_ref, dinv_ref, g_ref):
    i = pl.program_id(0)
    rowid = lax.broadcasted_iota(jnp.int32, (bm, 1), 0) + i * bm
    deg = (dg_ref[0, :, 0:1] + dg_ref[1, :, 0:1]
           + jnp.where(rowid < n_real, 1.0, 0.0))
    dinv = jnp.where(deg > 0, lax.rsqrt(jnp.maximum(deg, 1e-12)), 0.0)
    dinv_ref[...] = jnp.broadcast_to(dinv, (bm, LANES))
    g = dinv * jnp.dot(x_ref[...], w_ref[...],
                       preferred_element_type=jnp.float32)
    for p in range(halves):
      g_ref[p, :, :] = g[:, p * FH:(p + 1) * FH]

  return pl.pallas_call(
      body,
      grid=(m // bm,),
      in_specs=[pl.BlockSpec((bm, kdim), lambda i: (i, 0)),
                pl.BlockSpec((kdim, h), lambda i: (0, 0)),
                pl.BlockSpec((NC, bm, LANES), lambda i: (0, i, 0))],
      out_specs=[pl.BlockSpec((bm, LANES), lambda i: (i, 0)),
                 pl.BlockSpec((halves, bm, FH), lambda i: (0, i, 0))],
      out_shape=[jax.ShapeDtypeStruct((m, LANES), jnp.float32),
                 jax.ShapeDtypeStruct((halves, m, FH), jnp.float32)],
  )(xp, w, degp)


def _layer2_in(s1, g1h, dinv, b1r, w2):
  halves, m, fh = s1.shape
  h = halves * fh
  c = w2.shape[1]
  bm = 1024

  def body(s_ref, g_ref, dv, b, w, g2_ref):
    d = dv[:, 0:1]
    zs = []
    for p in range(halves):
      t = s_ref[p, :, :] + g_ref[p, :, :]
      zs.append(jnp.maximum(d * t + b[0:1, p * fh:(p + 1) * fh], 0.0))
    z = jnp.concatenate(zs, axis=1)
    g2_ref[0, :, :] = d * jnp.dot(z, w[...],
                                  preferred_element_type=jnp.float32)

  return pl.pallas_call(
      body,
      grid=(m // bm,),
      in_specs=[pl.BlockSpec((halves, bm, fh), lambda i: (0, i, 0)),
                pl.BlockSpec((halves, bm, fh), lambda i: (0, i, 0)),
                pl.BlockSpec((bm, LANES), lambda i: (i, 0)),
                pl.BlockSpec((1, h), lambda i: (0, 0)),
                pl.BlockSpec((h, c), lambda i: (0, 0))],
      out_specs=pl.BlockSpec((1, bm, c), lambda i: (0, i, 0)),
      out_shape=jax.ShapeDtypeStruct((1, m, c), jnp.float32),
  )(s1, g1h, dinv, b1r, w2)


def _finish(s2, g2h, dinv, b2r, n_out):
  nc, _, m, c = s2.shape
  bm = 400 if n_out % 400 == 0 else None

  def body(s_ref, g_ref, dv, b, o_ref):
    d = dv[:, 0:1]
    o_ref[...] = (d * (s_ref[0, 0, :, :] + s_ref[1, 0, :, :]
                       + g_ref[0, :, :]) + b[0:1, :])

  if bm is None:
    bm, rows = 1024, m
  else:
    rows = n_out
  out = pl.pallas_call(
      body,
      grid=(rows // bm,),
      in_specs=[pl.BlockSpec((nc, 1, bm, c), lambda i: (0, 0, i, 0)),
                pl.BlockSpec((1, bm, c), lambda i: (0, i, 0)),
                pl.BlockSpec((bm, LANES), lambda i: (i, 0)),
                pl.BlockSpec((1, c), lambda i: (0, 0))],
      out_specs=pl.BlockSpec((bm, c), lambda i: (i, 0)),
      out_shape=jax.ShapeDtypeStruct((rows, c), jnp.float32),
  )(s2, g2h, dinv, b2r)
  return out[:n_out]


def _ceil_to(v, mult):
  return (v + mult - 1) // mult * mult


@jax.jit
def kernel(x, adjs, W1, b1, W2, b2):
  n, _ = x.shape
  h = W1.shape[1]
  c = W2.shape[1]
  e = adjs.shape[1]

  npad = _ceil_to(n + 1, NS * ZB)
  chunks = _ceil_to(e, NW * B) // (NW * B)
  epad = chunks * NW * B

  pad_idx = jnp.full((2, epad - e), n, jnp.int32)
  rc = jnp.concatenate([adjs.astype(jnp.int32), pad_idx],
                       axis=1).reshape(2, NW, chunks, B)
  xp = jnp.pad(x, ((0, npad - n), (0, 0)))

  degp = _deg_hist(rc, npad, chunks)
  dinv, g1h = _mm_scale(xp, W1, degp, n)
  s1 = _edge_agg_split(g1h, rc, npad, chunks)
  g2h = _layer2_in(s1, g1h, dinv, b1.reshape(1, h), W2)
  s2 = _edge_agg(g2h, rc, npad, c // FH, chunks)
  return _finish(s2, g2h, dinv, b2.reshape(1, c), n)
